# baseline (speedup 1.0000x reference)
import functools

import jax
import jax.numpy as jnp
from jax import lax
from jax.experimental import pallas as pl
from jax.experimental.pallas import tpu as pltpu
from jax.experimental.pallas import tpu_sc as plsc

N = 10000
E = 160000
D = 256
DH = 128
NC, NS, L = 2, 16, 16
CH = 128
EPT = E // NS
NCHUNK = 80
EPAD = NCHUNK * CH
NBUF = 3
NMAIN = NCHUNK - 2
ROWS_PT = N // NS

_mesh = plsc.VectorSubcoreMesh(
    core_axis_name="c", subcore_axis_name="s", num_cores=NC, num_subcores=NS)


@functools.partial(
    pl.kernel,
    out_type=jax.ShapeDtypeStruct((NC, N, DH), jnp.bfloat16),
    mesh=_mesh,
    compiler_params=pltpu.CompilerParams(use_tc_tiling_on_sc=False,
                                         needs_layout_passes=False),
    scratch_types=[
        pltpu.VMEM((NCHUNK, CH), jnp.int32),
        pltpu.VMEM((NCHUNK, 2, CH), jnp.int32),
        pltpu.VMEM((NBUF, CH, DH), jnp.bfloat16),
        pltpu.VMEM_SHARED((N, DH), jnp.bfloat16),
    ] + [pltpu.SemaphoreType.DMA] * (2 * NBUF),
)
def _sc_aggregate(xparts, src4, dv3, zeros, out,
                  src_v, dv_v, rows_v, acc, *sems):
    gsem = sems[0:NBUF]
    ssem = sems[NBUF:2 * NBUF]
    c = lax.axis_index("c")
    s = lax.axis_index("s")

    pltpu.sync_copy(zeros, acc.at[pl.ds(s * ROWS_PT, ROWS_PT)])

    pltpu.sync_copy(src4.at[c, s], src_v)
    pltpu.sync_copy(dv3.at[s], dv_v)

    plsc.subcore_barrier()

    def gather_copy(j, b):
        return pltpu.make_async_copy(
            xparts.at[src_v.at[j]], rows_v.at[b], gsem[b])

    def scatter_copy(j, b):
        return pltpu.make_async_copy(
            rows_v.at[b], acc.at[dv_v.at[j, 0]], ssem[b])

    def process_chunk(j, b):
        gather_copy(j, b).wait()
        jrow = jnp.full((L,), j, jnp.int32)
        one = jnp.full((L,), 1, jnp.int32)
        for e in range(CH):
            sp = plsc.bitcast(
                plsc.load_gather(dv_v, [jrow, one,
                                        jnp.full((L,), e, jnp.int32)]),
                jnp.float32)
            spb = plsc.pack(sp, sp, format=plsc.PackFormat.INTERLEAVED)
            for k in range(DH // (2 * L)):
                sl = pl.ds(k * 2 * L, 2 * L)
                rows_v[b, e, sl] = rows_v[b, e, sl] * spb
        drain = scatter_copy(j - 1, (b - 1) % NBUF).wait
        if b == 0:
            pl.when(j >= 1)(drain)
        else:
            drain()
        scatter_copy(j, b).start(add=True)

    gather_copy(0, 0).start()
    gather_copy(1, 1).start()

    @pl.loop(0, NMAIN, step=NBUF)
    def _outer(jj):
        for b in range(NBUF):
            j = jj + b
            process_chunk(j, b)
            bn = (b + 2) % NBUF
            @pl.when(jj < NCHUNK - 2 - b)
            def _prefetch():
                gather_copy(j + 2, bn).start()

    for j, b in ((NCHUNK - 2, (NCHUNK - 2) % NBUF),
                 (NCHUNK - 1, (NCHUNK - 1) % NBUF)):
        process_chunk(j, b)

    scatter_copy(NCHUNK - 1, (NCHUNK - 1) % NBUF).wait()

    plsc.subcore_barrier()

    pltpu.sync_copy(acc.at[pl.ds(s * ROWS_PT, ROWS_PT)],
                    out.at[c, pl.ds(s * ROWS_PT, ROWS_PT)])


BN = 1000


def _tc_body(sup_ref, x_ref, wa_ref, wb_ref, wc_ref, g_ref, b_ref, o_ref):
    acc = jnp.dot(sup_ref[0], wa_ref[...], preferred_element_type=jnp.float32)
    acc = acc + jnp.dot(sup_ref[1], wb_ref[...],
                        preferred_element_type=jnp.float32)
    acc = acc + jnp.dot(x_ref[...], wc_ref[...],
                        preferred_element_type=jnp.float32)
    mu = jnp.mean(acc, axis=-1, keepdims=True)
    d = acc - mu
    var = jnp.mean(d * d, axis=-1, keepdims=True)
    y = d * lax.rsqrt(var + 1e-5) * g_ref[...] + b_ref[...]
    o_ref[...] = jnp.maximum(y, 0.0)


_tc_dense = pl.pallas_call(
    _tc_body,
    grid=(N // BN,),
    in_specs=[
        pl.BlockSpec((NC, BN, DH), lambda i: (0, i, 0)),
        pl.BlockSpec((BN, D), lambda i: (i, 0)),
        pl.BlockSpec((DH, D), lambda i: (0, 0)),
        pl.BlockSpec((DH, D), lambda i: (0, 0)),
        pl.BlockSpec((D, D), lambda i: (0, 0)),
        pl.BlockSpec((1, D), lambda i: (0, 0)),
        pl.BlockSpec((1, D), lambda i: (0, 0)),
    ],
    out_specs=pl.BlockSpec((BN, D), lambda i: (i, 0)),
    out_shape=jax.ShapeDtypeStruct((N, D), jnp.float32),
)


def kernel(x, A_edge_vals, weight, gamma, beta, A_edge_index):
    pad = ((0, 0), (0, EPAD - EPT))
    src2 = jnp.pad(A_edge_index[0].astype(jnp.int32).reshape(NS, EPT), pad)
    dst2 = jnp.pad(A_edge_index[1].astype(jnp.int32).reshape(NS, EPT), pad)
    val2 = jnp.pad(
        lax.bitcast_convert_type(A_edge_vals, jnp.int32).reshape(NS, EPT), pad)
    src4 = jnp.stack([src2, src2 + N], axis=0).reshape(NC, NS, NCHUNK, CH)
    dv3 = jnp.stack([dst2.reshape(NS, NCHUNK, CH),
                     val2.reshape(NS, NCHUNK, CH)], axis=2)
    xparts = jnp.concatenate(
        [x[:, :DH], x[:, DH:]], axis=0).astype(jnp.bfloat16)
    zeros = jnp.zeros((ROWS_PT, DH), jnp.bfloat16)

    sup = _sc_aggregate(xparts, src4, dv3, zeros)

    wa = weight[:, :DH].T.astype(jnp.bfloat16)
    wb = weight[:, DH:2 * DH].T.astype(jnp.bfloat16)
    wc = weight[:, 2 * DH:].T
    return _tc_dense(sup, x, wa, wb, wc,
                     gamma.reshape(1, D), beta.reshape(1, D))


# speedup vs baseline: 1.1748x; 1.1748x over previous
import functools

import jax
import jax.numpy as jnp
from jax import lax
from jax.experimental import pallas as pl
from jax.experimental.pallas import tpu as pltpu
from jax.experimental.pallas import tpu_sc as plsc

N = 10000
E = 160000
D = 256
DH = 128
NC, NS, L = 2, 16, 16
CH = 80
EPT = E // NS
NCHUNK = EPT // CH
NBUF = 4
PD = 3
NMAIN = 124
ROWS_PT = N // NS

_mesh = plsc.VectorSubcoreMesh(
    core_axis_name="c", subcore_axis_name="s", num_cores=NC, num_subcores=NS)


@functools.partial(
    pl.kernel,
    out_type=jax.ShapeDtypeStruct((NC, N, DH), jnp.bfloat16),
    mesh=_mesh,
    compiler_params=pltpu.CompilerParams(use_tc_tiling_on_sc=False,
                                         needs_layout_passes=False),
    scratch_types=[
        pltpu.VMEM((NCHUNK, CH), jnp.int32),
        pltpu.VMEM((NBUF, 2, CH), jnp.int32),
        pltpu.VMEM((NBUF, CH, DH), jnp.bfloat16),
        pltpu.VMEM_SHARED((N, DH), jnp.bfloat16),
    ] + [pltpu.SemaphoreType.DMA] * (3 * NBUF),
)
def _sc_aggregate(xparts, src4, dv3, zeros, out,
                  src_v, dvr, rows_v, acc, *sems):
    gsem = sems[0:NBUF]
    dsem = sems[NBUF:2 * NBUF]
    ssem = sems[2 * NBUF:3 * NBUF]
    c = lax.axis_index("c")
    s = lax.axis_index("s")

    pltpu.sync_copy(zeros, acc.at[pl.ds(s * ROWS_PT, ROWS_PT)])

    pltpu.sync_copy(src4.at[c, s], src_v)

    plsc.subcore_barrier()

    def gather_copies(j, b):
        return (
            pltpu.make_async_copy(
                xparts.at[src_v.at[j]], rows_v.at[b], gsem[b]),
            pltpu.make_async_copy(dv3.at[s, j], dvr.at[b], dsem[b]),
        )

    def start_stage(j, b):
        for d in gather_copies(j, b):
            d.start()

    def scatter_copy(j, b):
        return pltpu.make_async_copy(
            rows_v.at[b], acc.at[dvr.at[b, 0]], ssem[b])

    def process_chunk(j, b):
        rcp, dcp = gather_copies(j, b)
        rcp.wait()
        dcp.wait()
        for e in range(CH):
            sp = plsc.bitcast(
                plsc.load_gather(
                    dvr, [jnp.full((L,), b, jnp.int32),
                          jnp.full((L,), 1, jnp.int32),
                          jnp.full((L,), e, jnp.int32)]),
                jnp.float32)
            spb = plsc.pack(sp, sp, format=plsc.PackFormat.INTERLEAVED)
            for k in range(DH // (2 * L)):
                sl = pl.ds(k * 2 * L, 2 * L)
                rows_v[b, e, sl] = rows_v[b, e, sl] * spb
        drain = scatter_copy(j - 1, (b - 1) % NBUF).wait
        if b == 0:
            pl.when(j >= 1)(drain)
        else:
            drain()
        scatter_copy(j, b).start(add=True)

    for j0 in range(PD):
        start_stage(j0, j0)

    @pl.loop(0, NMAIN, step=NBUF)
    def _outer(jj):
        for b in range(NBUF):
            j = jj + b
            process_chunk(j, b)
            bn = (b + PD) % NBUF
            @pl.when(jj < NCHUNK - PD - b)
            def _prefetch():
                start_stage(j + PD, bn)

    for j in range(NMAIN, NCHUNK):
        process_chunk(j, j % NBUF)

    scatter_copy(NCHUNK - 1, (NCHUNK - 1) % NBUF).wait()

    plsc.subcore_barrier()

    pltpu.sync_copy(acc.at[pl.ds(s * ROWS_PT, ROWS_PT)],
                    out.at[c, pl.ds(s * ROWS_PT, ROWS_PT)])


BN = 1000


def _tc_body(sup_ref, x_ref, wa_ref, wb_ref, wc_ref, g_ref, b_ref, o_ref):
    acc = jnp.dot(sup_ref[0], wa_ref[...], preferred_element_type=jnp.float32)
    acc = acc + jnp.dot(sup_ref[1], wb_ref[...],
                        preferred_element_type=jnp.float32)
    acc = acc + jnp.dot(x_ref[...], wc_ref[...],
                        preferred_element_type=jnp.float32)
    mu = jnp.mean(acc, axis=-1, keepdims=True)
    d = acc - mu
    var = jnp.mean(d * d, axis=-1, keepdims=True)
    y = d * lax.rsqrt(var + 1e-5) * g_ref[...] + b_ref[...]
    o_ref[...] = jnp.maximum(y, 0.0)


_tc_dense = pl.pallas_call(
    _tc_body,
    grid=(N // BN,),
    in_specs=[
        pl.BlockSpec((NC, BN, DH), lambda i: (0, i, 0)),
        pl.BlockSpec((BN, D), lambda i: (i, 0)),
        pl.BlockSpec((DH, D), lambda i: (0, 0)),
        pl.BlockSpec((DH, D), lambda i: (0, 0)),
        pl.BlockSpec((D, D), lambda i: (0, 0)),
        pl.BlockSpec((1, D), lambda i: (0, 0)),
        pl.BlockSpec((1, D), lambda i: (0, 0)),
    ],
    out_specs=pl.BlockSpec((BN, D), lambda i: (i, 0)),
    out_shape=jax.ShapeDtypeStruct((N, D), jnp.float32),
)


def kernel(x, A_edge_vals, weight, gamma, beta, A_edge_index):
    src = A_edge_index[0].astype(jnp.int32)
    src4 = jnp.stack([src, src + N]).reshape(NC, NS, NCHUNK, CH)
    dst3 = A_edge_index[1].astype(jnp.int32).reshape(NS, NCHUNK, CH)
    vals3 = lax.bitcast_convert_type(A_edge_vals, jnp.int32).reshape(
        NS, NCHUNK, CH)
    dv3 = jnp.stack([dst3, vals3], axis=2)
    xparts = jnp.concatenate([x[:, :DH], x[:, DH:]], axis=0).astype(jnp.bfloat16)
    zeros = jnp.zeros((ROWS_PT, DH), jnp.bfloat16)

    sup = _sc_aggregate(xparts, src4, dv3, zeros)

    wa = weight[:, :DH].T.astype(jnp.bfloat16)
    wb = weight[:, DH:2 * DH].T.astype(jnp.bfloat16)
    wc = weight[:, 2 * DH:].T
    return _tc_dense(sup, x, wa, wb, wc,
                     gamma.reshape(1, D), beta.reshape(1, D))
